# Initial kernel scaffold; baseline (speedup 1.0000x reference)
import jax
import jax.numpy as jnp
from jax.experimental import pallas as pl


def kernel(input_ids, token_table, pos_table):
    raise NotImplementedError("write your pallas kernel here")


# speedup vs baseline: 1.0356x; 1.0356x over previous
import functools

import jax
import jax.numpy as jnp
from jax import lax
from jax.experimental import pallas as pl
from jax.experimental.pallas import tpu as pltpu
from jax.experimental.pallas import tpu_sc as plsc

HIDDEN = 768
BATCH = 4
SEQ = 2048
NC = 2
NS = 16
NW = NC * NS
TOTAL = BATCH * SEQ
RPW = TOTAL // NW
CHUNK = 64
NCHUNK = RPW // CHUNK
LANES = 16
NSLICE = HIDDEN // LANES


def _emb_body(idx_hbm, pos_hbm, tab_hbm, out_hbm, idx_v, rows_v, pos_v, gsem, psem):
    wid = lax.axis_index("s") * NC + lax.axis_index("c")
    base = wid * RPW
    pos_base = lax.rem(base, SEQ)
    pltpu.sync_copy(idx_hbm.at[wid], idx_v)
    for c in range(NCHUNK):
        gather = pltpu.async_copy(tab_hbm.at[idx_v.at[c]], rows_v, gsem)
        pcopy = pltpu.async_copy(
            pos_hbm.at[pl.ds(pos_base + c * CHUNK, CHUNK)], pos_v, psem)
        gather.wait()
        pcopy.wait()

        def add_row(r, carry):
            for j in range(NSLICE):
                sl = pl.ds(j * LANES, LANES)
                rows_v[r, sl] += pos_v[r, sl]
            return carry

        lax.fori_loop(0, CHUNK, add_row, 0)
        pltpu.sync_copy(rows_v, out_hbm.at[pl.ds(base + c * CHUNK, CHUNK)])


@jax.jit
def _emb(idx, token_table, pos_table):
    mesh = plsc.VectorSubcoreMesh(core_axis_name="c", subcore_axis_name="s")
    f = pl.kernel(
        _emb_body,
        mesh=mesh,
        out_type=jax.ShapeDtypeStruct((TOTAL, HIDDEN), jnp.float32),
        scratch_types=[
            pltpu.VMEM((NCHUNK, CHUNK), jnp.int32),
            pltpu.VMEM((CHUNK, HIDDEN), jnp.float32),
            pltpu.VMEM((CHUNK, HIDDEN), jnp.float32),
            pltpu.SemaphoreType.DMA,
            pltpu.SemaphoreType.DMA,
        ],
    )
    return f(idx, pos_table, token_table)


def kernel(input_ids, token_table, pos_table):
    idx = input_ids.reshape(NW, NCHUNK, CHUNK).astype(jnp.int32)
    out = _emb(idx, token_table, pos_table)
    return out.reshape(BATCH, SEQ, HIDDEN)
